# baseline (speedup 1.0000x reference)
import jax
import jax.numpy as jnp
from jax.experimental import pallas as pl
from jax.experimental.pallas import tpu as pltpu

_B = 3
_S = 2048
_E = 1024
_HD = 64
_BS = 16
_HALF = _S // 2
_SCALE = _HD ** -0.5


def _paged_attn_kernel(x1_ref, wq_ref, wk_ref, wv_ref, x_hbm, out_hbm,
                       x2buf, obuf, sem2, osem):
    t = pl.program_id(0)

    _Q = _HALF // 2

    def copy2_full(b):
        return pltpu.make_async_copy(
            x_hbm.at[pl.ds(b * _S + _HALF, _HALF), :],
            x2buf.at[b, pl.ds(0, _HALF), :],
            sem2.at[b])

    def copy2_q(j):
        return pltpu.make_async_copy(
            x_hbm.at[pl.ds(2 * _S + _HALF + j * _Q, _Q), :],
            x2buf.at[2, pl.ds(j * _Q, _Q), :],
            sem2.at[2 + j])

    def ocopy(b):
        return pltpu.make_async_copy(
            obuf.at[b % 2], out_hbm.at[pl.ds(b * _S, _S), :], osem.at[b % 2])

    @pl.when(t == 0)
    def _warmup():
        copy2_full(0).start()
        copy2_full(1).start()
        copy2_q(0).start()
        copy2_q(1).start()

    dn_nt = (((1,), (1,)), ((), ()))
    dn_nn = (((1,), (0,)), ((), ()))

    x16 = x1_ref[pl.ds(0, _BS), :]
    k16 = jax.lax.dot_general(x16, wk_ref[:, :], dn_nt,
                              preferred_element_type=jnp.float32)
    v16 = jax.lax.dot_general(x16, wv_ref[:, :], dn_nt,
                              preferred_element_type=jnp.float32)
    a = jax.lax.dot_general(k16 * _SCALE, wq_ref[:, :], dn_nn,
                            preferred_element_type=jnp.float32)

    def half(x_tile, base_d, nrows=_HALF):
        s_t = jax.lax.dot_general(a, x_tile, dn_nt,
                                  preferred_element_type=jnp.float32)
        row = jax.lax.broadcasted_iota(jnp.int32, (_BS, nrows), 1)
        m = jax.lax.broadcasted_iota(jnp.int32, (_BS, nrows), 0)
        d = base_d + (row >> 4)
        r = row & (_BS - 1)
        cnt = d.astype(jnp.float32) + (m <= r).astype(jnp.float32)
        smax = jnp.max(s_t, axis=0, keepdims=True)
        w = cnt * jnp.exp(s_t - smax)
        w = w / jnp.sum(w, axis=0, keepdims=True)
        return jax.lax.dot_general(w, v16, (((0,), (0,)), ((), ())),
                                   preferred_element_type=jnp.float32)

    out1 = half(x1_ref[:, :], 0)

    @pl.when(t >= 2)
    def _reclaim():
        ocopy(t - 2).wait()
    obuf[t % 2, pl.ds(0, _HALF), :] = out1

    @pl.when(t < 2)
    def _full_second_half():
        copy2_full(t).wait()
        out2 = half(x2buf[t], _HALF // _BS)
        obuf[t % 2, pl.ds(_HALF, _HALF), :] = out2

    @pl.when(t == 2)
    def _split_second_half():
        copy2_q(0).wait()
        out2a = half(x2buf[2, pl.ds(0, _Q), :], _HALF // _BS, _Q)
        obuf[t % 2, pl.ds(_HALF, _Q), :] = out2a
        copy2_q(1).wait()
        out2b = half(x2buf[2, pl.ds(_Q, _Q), :], (_HALF + _Q) // _BS, _Q)
        obuf[t % 2, pl.ds(_HALF + _Q, _Q), :] = out2b

    ocopy(t).start()

    @pl.when(t == _B - 1)
    def _drain():
        ocopy(_B - 2).wait()
        ocopy(_B - 1).wait()


@jax.jit
def kernel(x, Wq, Wk, Wv):
    xf = x.reshape(_B * _S, _E)
    out = pl.pallas_call(
        _paged_attn_kernel,
        grid=(_B,),
        in_specs=[
            pl.BlockSpec((_HALF, _E), lambda t: (2 * t, 0)),
            pl.BlockSpec((_HD, _E), lambda t: (0, 0)),
            pl.BlockSpec((_HD, _E), lambda t: (0, 0)),
            pl.BlockSpec((_HD, _E), lambda t: (0, 0)),
            pl.BlockSpec(memory_space=pltpu.MemorySpace.HBM),
        ],
        out_specs=pl.BlockSpec(memory_space=pltpu.MemorySpace.HBM),
        out_shape=jax.ShapeDtypeStruct((_B * _S, _HD), jnp.float32),
        scratch_shapes=[
            pltpu.VMEM((_B, _HALF, _E), jnp.float32),
            pltpu.VMEM((2, _S, _HD), jnp.float32),
            pltpu.SemaphoreType.DMA((4,)),
            pltpu.SemaphoreType.DMA((2,)),
        ],
    )(xf, Wq, Wk, Wv, xf)
    return out.reshape(_B, _S, _HD)


# speedup vs baseline: 1.0134x; 1.0134x over previous
import jax
import jax.numpy as jnp
from jax.experimental import pallas as pl
from jax.experimental.pallas import tpu as pltpu

_B = 3
_S = 2048
_E = 1024
_HD = 64
_BS = 16
_HALF = _S // 2
_SCALE = _HD ** -0.5


def _paged_attn_kernel(x1_ref, x1b_ref, wq_ref, wk_ref, wv_ref, x_hbm, out_hbm,
                       x2buf, obuf, sem2, osem):
    t = pl.program_id(0)

    def copy2(b):
        return pltpu.make_async_copy(
            x_hbm.at[pl.ds(b * _S + _HALF, _HALF), :], x2buf.at[b],
            sem2.at[b])

    def ocopy(b):
        return pltpu.make_async_copy(
            obuf.at[b % 2], out_hbm.at[pl.ds(b * _S, _S), :], osem.at[b % 2])

    @pl.when(t == 0)
    def _warmup():
        for b in range(_B):
            copy2(b).start()

    dn_nt = (((1,), (1,)), ((), ()))
    dn_nn = (((1,), (0,)), ((), ()))

    x16 = x1_ref[pl.ds(0, _BS), :]
    k16 = jax.lax.dot_general(x16, wk_ref[:, :], dn_nt,
                              preferred_element_type=jnp.float32)
    v16 = jax.lax.dot_general(x16, wv_ref[:, :], dn_nt,
                              preferred_element_type=jnp.float32)
    a = jax.lax.dot_general(k16 * _SCALE, wq_ref[:, :], dn_nn,
                            preferred_element_type=jnp.float32)

    def half(x_tile, base_d, nrows=_HALF):
        s_t = jax.lax.dot_general(a, x_tile, dn_nt,
                                  preferred_element_type=jnp.float32)
        row = jax.lax.broadcasted_iota(jnp.int32, (_BS, nrows), 1)
        m = jax.lax.broadcasted_iota(jnp.int32, (_BS, nrows), 0)
        d = base_d + (row >> 4)
        r = row & (_BS - 1)
        cnt = d.astype(jnp.float32) + (m <= r).astype(jnp.float32)
        smax = jnp.max(s_t, axis=0, keepdims=True)
        w = cnt * jnp.exp(s_t - smax)
        w = w / jnp.sum(w, axis=0, keepdims=True)
        return jax.lax.dot_general(w, v16, (((0,), (0,)), ((), ())),
                                   preferred_element_type=jnp.float32)

    out1 = half(x1_ref[:, :], 0, _HALF // 2)
    out1b = half(x1b_ref[:, :], _HALF // 2 // _BS, _HALF // 2)

    @pl.when(t >= 2)
    def _reclaim():
        ocopy(t - 2).wait()
    obuf[t % 2, pl.ds(0, _HALF // 2), :] = out1
    obuf[t % 2, pl.ds(_HALF // 2, _HALF // 2), :] = out1b

    copy2(t).wait()
    out2 = half(x2buf[t], _HALF // _BS)
    obuf[t % 2, pl.ds(_HALF, _HALF), :] = out2
    ocopy(t).start()

    @pl.when(t == _B - 1)
    def _drain():
        ocopy(_B - 2).wait()
        ocopy(_B - 1).wait()


@jax.jit
def kernel(x, Wq, Wk, Wv):
    xf = x.reshape(_B * _S, _E)
    out = pl.pallas_call(
        _paged_attn_kernel,
        grid=(_B,),
        in_specs=[
            pl.BlockSpec((_HALF // 2, _E), lambda t: (4 * t, 0)),
            pl.BlockSpec((_HALF // 2, _E), lambda t: (4 * t + 1, 0)),
            pl.BlockSpec((_HD, _E), lambda t: (0, 0)),
            pl.BlockSpec((_HD, _E), lambda t: (0, 0)),
            pl.BlockSpec((_HD, _E), lambda t: (0, 0)),
            pl.BlockSpec(memory_space=pltpu.MemorySpace.HBM),
        ],
        out_specs=pl.BlockSpec(memory_space=pltpu.MemorySpace.HBM),
        out_shape=jax.ShapeDtypeStruct((_B * _S, _HD), jnp.float32),
        scratch_shapes=[
            pltpu.VMEM((_B, _HALF, _E), jnp.float32),
            pltpu.VMEM((2, _S, _HD), jnp.float32),
            pltpu.SemaphoreType.DMA((_B,)),
            pltpu.SemaphoreType.DMA((2,)),
        ],
    )(xf, xf, Wq, Wk, Wv, xf)
    return out.reshape(_B, _S, _HD)


# speedup vs baseline: 1.0360x; 1.0223x over previous
import jax
import jax.numpy as jnp
from jax.experimental import pallas as pl
from jax.experimental.pallas import tpu as pltpu

_B = 3
_S = 2048
_E = 1024
_HD = 64
_BS = 16
_HALF = _S // 2
_SCALE = _HD ** -0.5


def _paged_attn_kernel(x1_ref, wq_ref, wk_ref, wv_ref, x_hbm, out_hbm,
                       x2buf, obuf, sem2, osem):
    t = pl.program_id(0)

    def copy2(b):
        return pltpu.make_async_copy(
            x_hbm.at[pl.ds(b * _S + _HALF, _HALF), :], x2buf.at[b],
            sem2.at[b])

    def ocopy(b):
        return pltpu.make_async_copy(
            obuf.at[b % 2], out_hbm.at[pl.ds(b * _S, _S), :], osem.at[b % 2])

    @pl.when(t == 0)
    def _warmup():
        for b in range(_B):
            copy2(b).start()

    dn_nt = (((1,), (1,)), ((), ()))
    dn_nn = (((1,), (0,)), ((), ()))

    x16 = x1_ref[pl.ds(0, _BS), :]
    k16 = jax.lax.dot_general(x16, wk_ref[:, :], dn_nt,
                              preferred_element_type=jnp.float32)
    v16 = jax.lax.dot_general(x16, wv_ref[:, :], dn_nt,
                              preferred_element_type=jnp.float32)
    a = jax.lax.dot_general(k16 * _SCALE, wq_ref[:, :], dn_nn,
                            preferred_element_type=jnp.float32)

    def half(x_tile, base_d):
        s_t = jax.lax.dot_general(a, x_tile, dn_nt,
                                  preferred_element_type=jnp.float32)
        row = jax.lax.broadcasted_iota(jnp.int32, (_BS, _HALF), 1)
        m = jax.lax.broadcasted_iota(jnp.int32, (_BS, _HALF), 0)
        d = base_d + (row >> 4)
        r = row & (_BS - 1)
        cnt = d.astype(jnp.float32) + (m <= r).astype(jnp.float32)
        smax = jnp.max(s_t, axis=0, keepdims=True)
        w = cnt * jnp.exp(s_t - smax)
        w = w / jnp.sum(w, axis=0, keepdims=True)
        return jax.lax.dot_general(w, v16, (((0,), (0,)), ((), ())),
                                   preferred_element_type=jnp.float32)

    out1 = half(x1_ref[:, :], 0)

    @pl.when(t >= 2)
    def _reclaim():
        ocopy(t - 2).wait()
    obuf[t % 2, pl.ds(0, _HALF), :] = out1

    copy2(t).wait()
    out2 = half(x2buf[t], _HALF // _BS)
    obuf[t % 2, pl.ds(_HALF, _HALF), :] = out2
    ocopy(t).start()

    @pl.when(t == _B - 1)
    def _drain():
        ocopy(_B - 2).wait()
        ocopy(_B - 1).wait()


@jax.jit
def kernel(x, Wq, Wk, Wv):
    xf = x.reshape(_B * _S, _E)
    out = pl.pallas_call(
        _paged_attn_kernel,
        grid=(_B,),
        in_specs=[
            pl.BlockSpec((_HALF, _E), lambda t: (2 * t, 0)),
            pl.BlockSpec((_HD, _E), lambda t: (0, 0)),
            pl.BlockSpec((_HD, _E), lambda t: (0, 0)),
            pl.BlockSpec((_HD, _E), lambda t: (0, 0)),
            pl.BlockSpec(memory_space=pltpu.MemorySpace.HBM),
        ],
        out_specs=pl.BlockSpec(memory_space=pltpu.MemorySpace.HBM),
        out_shape=jax.ShapeDtypeStruct((_B * _S, _HD), jnp.float32),
        scratch_shapes=[
            pltpu.VMEM((_B, _HALF, _E), jnp.float32),
            pltpu.VMEM((2, _S, _HD), jnp.float32),
            pltpu.SemaphoreType.DMA((_B,)),
            pltpu.SemaphoreType.DMA((2,)),
        ],
    )(xf, Wq, Wk, Wv, xf)
    return out.reshape(_B, _S, _HD)
